# baseline (speedup 1.0000x reference)
import functools
import numpy as np
import jax
import jax.numpy as jnp
from jax import lax
from jax.experimental import pallas as pl
from jax.experimental.pallas import tpu as pltpu, tpu_sc as plsc

NIN = 1024
NOUT = 1024
DEPTH = 10
SPLIT = 8
NSH = 2 ** SPLIT
BLK = 1024

_n = np.arange(NSH)
_d = np.where(_n < NSH - 1, np.floor(np.log2(_n + 1)).astype(np.int64), 0).astype(np.int32)
_SH_np = np.where(_n < NSH - 1, DEPTH - _d, 0).astype(np.int32)
_R_np = np.where(_n < NSH - 1, _n + 1 - (1 << _d), -1).astype(np.int32)
_E_np = np.zeros((16, NSH), np.float32)
_E_np[_d[:NSH - 1], _n[:NSH - 1]] = 1.0


def _tc_body(x_ref, w_ref, e_ref, sh_ref, r_ref, ytab_ref, o_ref, wo_ref, io_ref):
    xb = x_ref[...]
    lam = jnp.dot(xb, w_ref[...], preferred_element_type=jnp.float32)
    bits = (lam > 0).astype(jnp.int32)
    col = lax.broadcasted_iota(jnp.int32, (1, 16), 1)
    pw = jnp.where(col < DEPTH, lax.shift_right_logical(512, col), 0)
    p = jnp.sum(bits * pw, axis=1, keepdims=True)
    lsel = jnp.dot(lam, e_ref[...], preferred_element_type=jnp.float32)
    t = lax.shift_right_logical(p, sh_ref[...])
    S = jnp.where(t == r_ref[...], lsel, 0.0).astype(jnp.bfloat16)
    o_ref[...] = jnp.dot(S, ytab_ref[...], preferred_element_type=jnp.float32)
    col32 = lax.broadcasted_iota(jnp.int32, (1, 32), 1)
    wo_ref[...] = jnp.where(col32 < 16, lam[:, 8:9], lam[:, 9:10])
    idx8 = lax.shift_right_logical(p, 2) + 255
    idx9 = lax.shift_right_logical(p, 1) + 511
    col2 = lax.broadcasted_iota(jnp.int32, (1, 2), 1)
    io_ref[...] = jnp.where(col2 == 0, idx8, idx9)


def _tc_call(x2, wT, ytab_s, nb):
    e = jnp.asarray(_E_np)
    sh = jnp.asarray(_SH_np).reshape(1, NSH)
    r = jnp.asarray(_R_np).reshape(1, NSH)
    return pl.pallas_call(
        _tc_body,
        grid=(nb // BLK,),
        in_specs=[
            pl.BlockSpec((BLK, NIN), lambda i: (i, 0)),
            pl.BlockSpec((NIN, 16), lambda i: (0, 0)),
            pl.BlockSpec((16, NSH), lambda i: (0, 0)),
            pl.BlockSpec((1, NSH), lambda i: (0, 0)),
            pl.BlockSpec((1, NSH), lambda i: (0, 0)),
            pl.BlockSpec((NSH, NOUT), lambda i: (0, 0)),
        ],
        out_specs=[
            pl.BlockSpec((BLK, NOUT), lambda i: (i, 0)),
            pl.BlockSpec((BLK, 32), lambda i: (i, 0)),
            pl.BlockSpec((BLK, 2), lambda i: (i, 0)),
        ],
        out_shape=[
            jax.ShapeDtypeStruct((nb, NOUT), jnp.float32),
            jax.ShapeDtypeStruct((nb, 32), jnp.float32),
            jax.ShapeDtypeStruct((nb, 2), jnp.int32),
        ],
    )(x2, wT, e, sh, r, ytab_s)


_NW = 32
_TPW = 256
_CH = 16


_NCHUNK = _TPW // _CH


def _sc_body(ys_hbm, w_hbm, idx_hbm, ytab_hbm, out_hbm,
             w_v, idx_v, idxc0, idxc1, rows0, rows1, acc0, acc1,
             semg0, semg1, sema0, sema1, semo0, semo1):
    wid = lax.axis_index("s") * 2 + lax.axis_index("c")
    base = wid * _TPW
    pltpu.sync_copy(w_hbm.at[pl.ds(base * 32, _TPW * 32)], w_v)
    pltpu.sync_copy(idx_hbm.at[pl.ds(base * 2, _TPW * 2)], idx_v)

    idxc = (idxc0, idxc1)
    rows = (rows0, rows1)
    acc = (acc0, acc1)
    semg = (semg0, semg1)
    sema = (sema0, sema1)
    semo = (semo0, semo1)

    def issue(c, b):
        idxc[b][pl.ds(0, 16)] = idx_v[pl.ds(c * 32, 16)]
        idxc[b][pl.ds(16, 16)] = idx_v[pl.ds(c * 32 + 16, 16)]
        pltpu.async_copy(ytab_hbm.at[idxc[b]], rows[b], semg[b])
        pltpu.async_copy(ys_hbm.at[pl.ds(base + c * _CH, _CH)], acc[b], sema[b])

    def wait_in(c, b):
        pltpu.make_async_copy(ytab_hbm.at[idxc[b]], rows[b], semg[b]).wait()
        pltpu.make_async_copy(ys_hbm.at[pl.ds(base + c * _CH, _CH)], acc[b],
                              sema[b]).wait()

    def wait_out(c, b):
        pltpu.make_async_copy(acc[b], out_hbm.at[pl.ds(base + c * _CH, _CH)],
                              semo[b]).wait()

    def combine(c, b):
        def token(t, carry2):
            w8 = w_v[pl.ds((c * _CH + t) * 32, 16)]
            w9 = w_v[pl.ds((c * _CH + t) * 32 + 16, 16)]
            for j in range(NOUT // 16):
                sl = pl.ds(j * 16, 16)
                val = w8 * rows[b][2 * t, sl] + w9 * rows[b][2 * t + 1, sl]
                acc[b][t, sl] = acc[b][t, sl] + val
            return carry2

        lax.fori_loop(0, _CH, token, 0)
        pltpu.async_copy(acc[b], out_hbm.at[pl.ds(base + c * _CH, _CH)], semo[b])

    issue(0, 0)

    def pair(c2, carry):
        c = 2 * c2
        @pl.when(c2 > 0)
        def _():
            wait_out(c - 1, 1)
        issue(c + 1, 1)
        wait_in(c, 0)
        combine(c, 0)
        @pl.when(c2 < _NCHUNK // 2 - 1)
        def _():
            wait_out(c, 0)
            issue(c + 2, 0)
        wait_in(c + 1, 1)
        combine(c + 1, 1)
        return carry

    lax.fori_loop(0, _NCHUNK // 2, pair, 0)
    wait_out(_NCHUNK - 2, 0)
    wait_out(_NCHUNK - 1, 1)


def kernel(x, W_sel, Y):
    orig_shape = x.shape
    x2 = x.reshape(-1, NIN) if x.ndim == 3 else x
    nb = x2.shape[0]
    wT = jnp.zeros((NIN, 16), jnp.float32).at[:, :DEPTH].set(W_sel.T)
    ytab_s = jnp.concatenate([Y[:NSH - 1], jnp.zeros((1, NOUT), Y.dtype)],
                             axis=0).astype(jnp.bfloat16)

    y_shallow, w_bcast, idx_pair = _tc_call(x2, wT, ytab_s, nb)

    mesh = plsc.VectorSubcoreMesh(core_axis_name="c", subcore_axis_name="s")
    sc = functools.partial(
        pl.kernel,
        out_type=jax.ShapeDtypeStruct((nb, NOUT), jnp.float32),
        mesh=mesh,
        scratch_types=[
            pltpu.VMEM((_TPW * 32,), jnp.float32),
            pltpu.VMEM((_TPW * 2,), jnp.int32),
            pltpu.VMEM((2 * _CH,), jnp.int32),
            pltpu.VMEM((2 * _CH,), jnp.int32),
            pltpu.VMEM((2 * _CH, NOUT), jnp.float32),
            pltpu.VMEM((2 * _CH, NOUT), jnp.float32),
            pltpu.VMEM((_CH, NOUT), jnp.float32),
            pltpu.VMEM((_CH, NOUT), jnp.float32),
            pltpu.SemaphoreType.DMA,
            pltpu.SemaphoreType.DMA,
            pltpu.SemaphoreType.DMA,
            pltpu.SemaphoreType.DMA,
            pltpu.SemaphoreType.DMA,
            pltpu.SemaphoreType.DMA,
        ],
    )(_sc_body)
    y = sc(y_shallow, w_bcast.reshape(-1), idx_pair.reshape(-1), Y)

    if orig_shape[1] != NIN:
        y = y.reshape(orig_shape[0], orig_shape[1], NOUT)
    return y


# speedup vs baseline: 1.7338x; 1.7338x over previous
import functools
import numpy as np
import jax
import jax.numpy as jnp
from jax import lax
from jax.experimental import pallas as pl
from jax.experimental.pallas import tpu as pltpu, tpu_sc as plsc

NIN = 1024
NOUT = 1024
DEPTH = 10
SPLIT = 8
NSH = 2 ** SPLIT
BLK = 1024

_n = np.arange(NSH)
_d = np.where(_n < NSH - 1, np.floor(np.log2(_n + 1)).astype(np.int64), 0).astype(np.int32)
_SH_np = np.where(_n < NSH - 1, DEPTH - _d, 0).astype(np.int32)
_R_np = np.where(_n < NSH - 1, _n + 1 - (1 << _d), -1).astype(np.int32)
_E_np = np.zeros((16, NSH), np.float32)
_E_np[_d[:NSH - 1], _n[:NSH - 1]] = 1.0


def _tc_body(x_ref, w_ref, e_ref, sh_ref, r_ref, ytab_ref, o_ref, wo_ref, io_ref):
    xb = x_ref[...]
    lam = jnp.dot(xb, w_ref[...], preferred_element_type=jnp.float32)
    bits = (lam > 0).astype(jnp.int32)
    col = lax.broadcasted_iota(jnp.int32, (1, 16), 1)
    pw = jnp.where(col < DEPTH, lax.shift_right_logical(512, col), 0)
    p = jnp.sum(bits * pw, axis=1, keepdims=True)
    lsel = jnp.dot(lam, e_ref[...], preferred_element_type=jnp.float32)
    t = lax.shift_right_logical(p, sh_ref[...])
    S = jnp.where(t == r_ref[...], lsel, 0.0).astype(jnp.bfloat16)
    o_ref[...] = jnp.dot(S, ytab_ref[...], preferred_element_type=jnp.float32)
    col32 = lax.broadcasted_iota(jnp.int32, (1, 32), 1)
    wo_ref[...] = jnp.where(col32 < 16, lam[:, 8:9], lam[:, 9:10])
    idx8 = lax.shift_right_logical(p, 2) + 255
    idx9 = lax.shift_right_logical(p, 1) + 511
    col2 = lax.broadcasted_iota(jnp.int32, (1, 2), 1)
    io_ref[...] = jnp.where(col2 == 0, idx8, idx9)


def _tc_call(x2, wT, ytab_s, nb):
    e = jnp.asarray(_E_np)
    sh = jnp.asarray(_SH_np).reshape(1, NSH)
    r = jnp.asarray(_R_np).reshape(1, NSH)
    return pl.pallas_call(
        _tc_body,
        grid=(nb // BLK,),
        in_specs=[
            pl.BlockSpec((BLK, NIN), lambda i: (i, 0)),
            pl.BlockSpec((NIN, 16), lambda i: (0, 0)),
            pl.BlockSpec((16, NSH), lambda i: (0, 0)),
            pl.BlockSpec((1, NSH), lambda i: (0, 0)),
            pl.BlockSpec((1, NSH), lambda i: (0, 0)),
            pl.BlockSpec((NSH, NOUT), lambda i: (0, 0)),
        ],
        out_specs=[
            pl.BlockSpec((BLK, NOUT), lambda i: (i, 0)),
            pl.BlockSpec((BLK, 32), lambda i: (i, 0)),
            pl.BlockSpec((BLK, 2), lambda i: (i, 0)),
        ],
        out_shape=[
            jax.ShapeDtypeStruct((nb, NOUT), jnp.float32),
            jax.ShapeDtypeStruct((nb, 32), jnp.float32),
            jax.ShapeDtypeStruct((nb, 2), jnp.int32),
        ],
    )(x2, wT, e, sh, r, ytab_s)


_NW = 32
_TPW = 256
_CH = 16


_NCHUNK = _TPW // _CH


def _sc_body(ys_hbm, w_hbm, idx_hbm, ytab_hbm, out_hbm,
             w_v, idx_v, idxc0, idxc1, rows0, rows1, acc0, acc1,
             semg0, semg1, sema0, sema1, semo0, semo1):
    wid = lax.axis_index("s") * 2 + lax.axis_index("c")
    base = wid * _TPW
    pltpu.sync_copy(w_hbm.at[pl.ds(base * 32, _TPW * 32)], w_v)
    pltpu.sync_copy(idx_hbm.at[pl.ds(base * 2, _TPW * 2)], idx_v)

    idxc = (idxc0, idxc1)
    rows = (rows0, rows1)
    acc = (acc0, acc1)
    semg = (semg0, semg1)
    sema = (sema0, sema1)
    semo = (semo0, semo1)

    def issue(c, b):
        idxc[b][pl.ds(0, 16)] = idx_v[pl.ds(c * 32, 16)]
        idxc[b][pl.ds(16, 16)] = idx_v[pl.ds(c * 32 + 16, 16)]
        pltpu.async_copy(ytab_hbm.at[idxc[b]], rows[b], semg[b])
        pltpu.async_copy(ys_hbm.at[pl.ds(base + c * _CH, _CH)], acc[b], sema[b])

    def wait_in(c, b):
        pltpu.make_async_copy(ytab_hbm.at[idxc[b]], rows[b], semg[b]).wait()
        pltpu.make_async_copy(ys_hbm.at[pl.ds(base + c * _CH, _CH)], acc[b],
                              sema[b]).wait()

    def wait_out(c, b):
        pltpu.make_async_copy(acc[b], out_hbm.at[pl.ds(base + c * _CH, _CH)],
                              semo[b]).wait()

    def combine(c, b):
        def token(t, carry2):
            w8 = w_v[pl.ds((c * _CH + t) * 32, 16)]
            w9 = w_v[pl.ds((c * _CH + t) * 32 + 16, 16)]
            for g in range(NOUT // 16 // 8):
                vals = []
                for k in range(8):
                    sl = pl.ds((g * 8 + k) * 16, 16)
                    vals.append(w8 * rows[b][2 * t, sl]
                                + w9 * rows[b][2 * t + 1, sl])
                for k in range(8):
                    sl = pl.ds((g * 8 + k) * 16, 16)
                    plsc.addupdate(acc[b].at[t, sl], vals[k])
            return carry2

        lax.fori_loop(0, _CH, token, 0)
        pltpu.async_copy(acc[b], out_hbm.at[pl.ds(base + c * _CH, _CH)], semo[b])

    issue(0, 0)

    def pair(c2, carry):
        c = 2 * c2
        @pl.when(c2 > 0)
        def _():
            wait_out(c - 1, 1)
        issue(c + 1, 1)
        wait_in(c, 0)
        combine(c, 0)
        @pl.when(c2 < _NCHUNK // 2 - 1)
        def _():
            wait_out(c, 0)
            issue(c + 2, 0)
        wait_in(c + 1, 1)
        combine(c + 1, 1)
        return carry

    lax.fori_loop(0, _NCHUNK // 2, pair, 0)
    wait_out(_NCHUNK - 2, 0)
    wait_out(_NCHUNK - 1, 1)


def kernel(x, W_sel, Y):
    orig_shape = x.shape
    x2 = x.reshape(-1, NIN) if x.ndim == 3 else x
    nb = x2.shape[0]
    wT = jnp.zeros((NIN, 16), jnp.float32).at[:, :DEPTH].set(W_sel.T)
    ytab_s = jnp.concatenate([Y[:NSH - 1], jnp.zeros((1, NOUT), Y.dtype)],
                             axis=0).astype(jnp.bfloat16)

    y_shallow, w_bcast, idx_pair = _tc_call(x2, wT, ytab_s, nb)

    mesh = plsc.VectorSubcoreMesh(core_axis_name="c", subcore_axis_name="s")
    sc = functools.partial(
        pl.kernel,
        out_type=jax.ShapeDtypeStruct((nb, NOUT), jnp.float32),
        mesh=mesh,
        scratch_types=[
            pltpu.VMEM((_TPW * 32,), jnp.float32),
            pltpu.VMEM((_TPW * 2,), jnp.int32),
            pltpu.VMEM((2 * _CH,), jnp.int32),
            pltpu.VMEM((2 * _CH,), jnp.int32),
            pltpu.VMEM((2 * _CH, NOUT), jnp.float32),
            pltpu.VMEM((2 * _CH, NOUT), jnp.float32),
            pltpu.VMEM((_CH, NOUT), jnp.float32),
            pltpu.VMEM((_CH, NOUT), jnp.float32),
            pltpu.SemaphoreType.DMA,
            pltpu.SemaphoreType.DMA,
            pltpu.SemaphoreType.DMA,
            pltpu.SemaphoreType.DMA,
            pltpu.SemaphoreType.DMA,
            pltpu.SemaphoreType.DMA,
        ],
    )(_sc_body)
    y = sc(y_shallow, w_bcast.reshape(-1), idx_pair.reshape(-1), Y)

    if orig_shape[1] != NIN:
        y = y.reshape(orig_shape[0], orig_shape[1], NOUT)
    return y


# speedup vs baseline: 1.7341x; 1.0002x over previous
import functools
import numpy as np
import jax
import jax.numpy as jnp
from jax import lax
from jax.experimental import pallas as pl
from jax.experimental.pallas import tpu as pltpu, tpu_sc as plsc

NIN = 1024
NOUT = 1024
DEPTH = 10
SPLIT = 8
NSH = 2 ** SPLIT
BLK = 1024

_n = np.arange(NSH)
_d = np.where(_n < NSH - 1, np.floor(np.log2(_n + 1)).astype(np.int64), 0).astype(np.int32)
_SH_np = np.where(_n < NSH - 1, DEPTH - _d, 0).astype(np.int32)
_R_np = np.where(_n < NSH - 1, _n + 1 - (1 << _d), -1).astype(np.int32)
_E_np = np.zeros((16, NSH), np.float32)
_E_np[_d[:NSH - 1], _n[:NSH - 1]] = 1.0


def _tc_body(x_ref, w_ref, e_ref, sh_ref, r_ref, ytab_ref, o_ref, wo_ref, io_ref):
    xb = x_ref[...]
    lam = jnp.dot(xb, w_ref[...], preferred_element_type=jnp.float32)
    bits = (lam > 0).astype(jnp.int32)
    col = lax.broadcasted_iota(jnp.int32, (1, 16), 1)
    pw = jnp.where(col < DEPTH, lax.shift_right_logical(512, col), 0)
    p = jnp.sum(bits * pw, axis=1, keepdims=True)
    lsel = jnp.dot(lam, e_ref[...], preferred_element_type=jnp.float32)
    t = lax.shift_right_logical(p, sh_ref[...])
    S = jnp.where(t == r_ref[...], lsel, 0.0).astype(jnp.bfloat16)
    o_ref[...] = jnp.dot(S, ytab_ref[...], preferred_element_type=jnp.float32)
    col32 = lax.broadcasted_iota(jnp.int32, (1, 32), 1)
    wo_ref[...] = jnp.where(col32 < 16, lam[:, 8:9], lam[:, 9:10])
    idx8 = lax.shift_right_logical(p, 2) + 255
    idx9 = lax.shift_right_logical(p, 1) + 511
    col2 = lax.broadcasted_iota(jnp.int32, (1, 2), 1)
    io_ref[...] = jnp.where(col2 == 0, idx8, idx9)


def _tc_call(x2, wT, ytab_s, nb):
    e = jnp.asarray(_E_np)
    sh = jnp.asarray(_SH_np).reshape(1, NSH)
    r = jnp.asarray(_R_np).reshape(1, NSH)
    return pl.pallas_call(
        _tc_body,
        grid=(nb // BLK,),
        in_specs=[
            pl.BlockSpec((BLK, NIN), lambda i: (i, 0)),
            pl.BlockSpec((NIN, 16), lambda i: (0, 0)),
            pl.BlockSpec((16, NSH), lambda i: (0, 0)),
            pl.BlockSpec((1, NSH), lambda i: (0, 0)),
            pl.BlockSpec((1, NSH), lambda i: (0, 0)),
            pl.BlockSpec((NSH, NOUT), lambda i: (0, 0)),
        ],
        out_specs=[
            pl.BlockSpec((BLK, NOUT), lambda i: (i, 0)),
            pl.BlockSpec((BLK, 32), lambda i: (i, 0)),
            pl.BlockSpec((BLK, 2), lambda i: (i, 0)),
        ],
        out_shape=[
            jax.ShapeDtypeStruct((nb, NOUT), jnp.float32),
            jax.ShapeDtypeStruct((nb, 32), jnp.float32),
            jax.ShapeDtypeStruct((nb, 2), jnp.int32),
        ],
    )(x2, wT, e, sh, r, ytab_s)


_NW = 32
_TPW = 256
_CH = 16


_NCHUNK = _TPW // _CH


def _sc_body(ys_hbm, w_hbm, idx_hbm, ytab_hbm, out_hbm,
             w_v, idx_v, idxc0, idxc1, rows0, rows1, acc0, acc1,
             semg0, semg1, sema0, sema1, semo0, semo1):
    wid = lax.axis_index("s") * 2 + lax.axis_index("c")
    base = wid * _TPW
    pltpu.sync_copy(w_hbm.at[pl.ds(base * 32, _TPW * 32)], w_v)
    pltpu.sync_copy(idx_hbm.at[pl.ds(base * 2, _TPW * 2)], idx_v)

    idxc = (idxc0, idxc1)
    rows = (rows0, rows1)
    acc = (acc0, acc1)
    semg = (semg0, semg1)
    sema = (sema0, sema1)
    semo = (semo0, semo1)

    def issue(c, b):
        idxc[b][pl.ds(0, 16)] = idx_v[pl.ds(c * 32, 16)]
        idxc[b][pl.ds(16, 16)] = idx_v[pl.ds(c * 32 + 16, 16)]
        pltpu.async_copy(ytab_hbm.at[idxc[b]], rows[b], semg[b])
        pltpu.async_copy(ys_hbm.at[pl.ds(base + c * _CH, _CH)], acc[b], sema[b])

    def wait_in(c, b):
        pltpu.make_async_copy(ytab_hbm.at[idxc[b]], rows[b], semg[b]).wait()
        pltpu.make_async_copy(ys_hbm.at[pl.ds(base + c * _CH, _CH)], acc[b],
                              sema[b]).wait()

    def wait_out(c, b):
        pltpu.make_async_copy(acc[b], out_hbm.at[pl.ds(base + c * _CH, _CH)],
                              semo[b]).wait()

    def combine(c, b):
        def token(t, carry2):
            w8 = w_v[pl.ds((c * _CH + t) * 32, 16)]
            w9 = w_v[pl.ds((c * _CH + t) * 32 + 16, 16)]
            for g in range(NOUT // 16 // 16):
                vals = []
                for k in range(16):
                    sl = pl.ds((g * 16 + k) * 16, 16)
                    vals.append(w8 * rows[b][2 * t, sl]
                                + w9 * rows[b][2 * t + 1, sl])
                for k in range(16):
                    sl = pl.ds((g * 16 + k) * 16, 16)
                    plsc.addupdate(acc[b].at[t, sl], vals[k])
            return carry2

        lax.fori_loop(0, _CH, token, 0)
        pltpu.async_copy(acc[b], out_hbm.at[pl.ds(base + c * _CH, _CH)], semo[b])

    issue(0, 0)

    def pair(c2, carry):
        c = 2 * c2
        @pl.when(c2 > 0)
        def _():
            wait_out(c - 1, 1)
        issue(c + 1, 1)
        wait_in(c, 0)
        combine(c, 0)
        @pl.when(c2 < _NCHUNK // 2 - 1)
        def _():
            wait_out(c, 0)
            issue(c + 2, 0)
        wait_in(c + 1, 1)
        combine(c + 1, 1)
        return carry

    lax.fori_loop(0, _NCHUNK // 2, pair, 0)
    wait_out(_NCHUNK - 2, 0)
    wait_out(_NCHUNK - 1, 1)


def kernel(x, W_sel, Y):
    orig_shape = x.shape
    x2 = x.reshape(-1, NIN) if x.ndim == 3 else x
    nb = x2.shape[0]
    wT = jnp.zeros((NIN, 16), jnp.float32).at[:, :DEPTH].set(W_sel.T)
    ytab_s = jnp.concatenate([Y[:NSH - 1], jnp.zeros((1, NOUT), Y.dtype)],
                             axis=0).astype(jnp.bfloat16)

    y_shallow, w_bcast, idx_pair = _tc_call(x2, wT, ytab_s, nb)

    mesh = plsc.VectorSubcoreMesh(core_axis_name="c", subcore_axis_name="s")
    sc = functools.partial(
        pl.kernel,
        out_type=jax.ShapeDtypeStruct((nb, NOUT), jnp.float32),
        mesh=mesh,
        scratch_types=[
            pltpu.VMEM((_TPW * 32,), jnp.float32),
            pltpu.VMEM((_TPW * 2,), jnp.int32),
            pltpu.VMEM((2 * _CH,), jnp.int32),
            pltpu.VMEM((2 * _CH,), jnp.int32),
            pltpu.VMEM((2 * _CH, NOUT), jnp.float32),
            pltpu.VMEM((2 * _CH, NOUT), jnp.float32),
            pltpu.VMEM((_CH, NOUT), jnp.float32),
            pltpu.VMEM((_CH, NOUT), jnp.float32),
            pltpu.SemaphoreType.DMA,
            pltpu.SemaphoreType.DMA,
            pltpu.SemaphoreType.DMA,
            pltpu.SemaphoreType.DMA,
            pltpu.SemaphoreType.DMA,
            pltpu.SemaphoreType.DMA,
        ],
    )(_sc_body)
    y = sc(y_shallow, w_bcast.reshape(-1), idx_pair.reshape(-1), Y)

    if orig_shape[1] != NIN:
        y = y.reshape(orig_shape[0], orig_shape[1], NOUT)
    return y
